# baseline (speedup 1.0000x reference)
import functools

import jax
import jax.numpy as jnp
from jax.experimental import pallas as pl
from jax.experimental.pallas import tpu as pltpu

DIM = 1024
HEADS = 16
DH = 64
E = 8
DFF = 1024
S = 2048

SBLK = 256
NSB = S // SBLK
NCHUNK = 2 * HEADS
CPAD = 128


def _qkv_router_body(x_ref, wqkv_ref, pool_ref, exg_ref, wg_ref,
                     qkv_ref, comb_ref):
    x = x_ref[...]
    x16 = x.astype(jnp.bfloat16)
    qkv = jnp.dot(x16, wqkv_ref[...], preferred_element_type=jnp.float32)
    qk = qkv[:, : 2 * HEADS * DH]
    v = qkv[:, 2 * HEADS * DH :]
    sq = (qk * qk).astype(jnp.bfloat16)
    ss = jnp.dot(sq, pool_ref[...], preferred_element_type=jnp.float32)
    rec = (1.0 / (jnp.sqrt(ss) + 1e-6)).astype(jnp.bfloat16)
    recb = jnp.dot(rec, exg_ref[...], preferred_element_type=jnp.float32)
    qkn = (qk * recb).astype(jnp.bfloat16)
    qkv_ref[...] = jnp.concatenate([qkn, v.astype(jnp.bfloat16)], axis=1)

    logits = jnp.dot(x, wg_ref[...], preferred_element_type=jnp.float32)
    eids = jax.lax.broadcasted_iota(jnp.int32, (SBLK, E), 1)
    m1 = jnp.max(logits, axis=-1, keepdims=True)
    a1 = jnp.argmax(logits, axis=-1)[:, None]
    masked = jnp.where(eids == a1, -jnp.inf, logits)
    m2 = jnp.max(masked, axis=-1, keepdims=True)
    a2 = jnp.argmax(masked, axis=-1)[:, None]
    z = jnp.exp(m2 - m1)
    g1 = 1.0 / (1.0 + z)
    g2 = z / (1.0 + z)
    comb_ref[...] = jnp.where(eids == a1, g1, 0.0) + jnp.where(eids == a2, g2, 0.0)


def _qkv_router(x, wqkv, pool, exg, wg):
    return pl.pallas_call(
        _qkv_router_body,
        grid=(NSB,),
        in_specs=[
            pl.BlockSpec((SBLK, DIM), lambda s: (s, 0)),
            pl.BlockSpec((DIM, 3 * HEADS * DH), lambda s: (0, 0)),
            pl.BlockSpec((2 * HEADS * DH, CPAD), lambda s: (0, 0)),
            pl.BlockSpec((CPAD, 2 * HEADS * DH), lambda s: (0, 0)),
            pl.BlockSpec((DIM, E), lambda s: (0, 0)),
        ],
        out_specs=[
            pl.BlockSpec((SBLK, 3 * HEADS * DH), lambda s: (s, 0)),
            pl.BlockSpec((SBLK, E), lambda s: (s, 0)),
        ],
        out_shape=[
            jax.ShapeDtypeStruct((S, 3 * HEADS * DH), jnp.bfloat16),
            jax.ShapeDtypeStruct((S, E), jnp.float32),
        ],
    )(x, wqkv, pool, exg, wg)


VAUG = 128


def _poly_exp(s):
    t = s * (1.0 / 24.0) + (1.0 / 6.0)
    t = t * s + 0.5
    t = t * s + 1.0
    return t * s + 1.0


def _attn_body(q_ref, k_ref, v_ref, o_ref):
    qb = pl.program_id(1)
    q = q_ref[0]

    def pblock(j, masked):
        k = k_ref[0, pl.ds(j * SBLK, SBLK), :]
        s = jax.lax.dot_general(q, k, (((1,), (1,)), ((), ())),
                                preferred_element_type=jnp.float32)
        p = _poly_exp(s)
        if masked:
            row = jax.lax.broadcasted_iota(jnp.int32, (SBLK, SBLK), 0)
            col = jax.lax.broadcasted_iota(jnp.int32, (SBLK, SBLK), 1)
            p = jnp.where(col <= row, p, 0.0)
        return p.astype(jnp.bfloat16)

    def step(j, acc):
        p = pblock(j, masked=False)
        vv = v_ref[0, pl.ds(j * SBLK, SBLK), :]
        return acc + jnp.dot(p, vv, preferred_element_type=jnp.float32)

    acc0 = jnp.zeros((SBLK, VAUG), jnp.float32)
    acc = jax.lax.fori_loop(0, qb, step, acc0)
    p = pblock(qb, masked=True)
    vv = v_ref[0, pl.ds(qb * SBLK, SBLK), :]
    acc = acc + jnp.dot(p, vv, preferred_element_type=jnp.float32)
    o_ref[0] = (acc[:, :DH] / acc[:, DH : DH + 1]).astype(jnp.bfloat16)


def _flash_attn(q, k, vaug):
    return pl.pallas_call(
        _attn_body,
        grid=(HEADS, NSB),
        in_specs=[
            pl.BlockSpec((1, SBLK, DH), lambda h, s: (h, s, 0)),
            pl.BlockSpec((1, S, DH), lambda h, s: (h, 0, 0)),
            pl.BlockSpec((1, S, VAUG), lambda h, s: (h, 0, 0)),
        ],
        out_specs=pl.BlockSpec((1, SBLK, DH), lambda h, s: (h, s, 0)),
        out_shape=jax.ShapeDtypeStruct((HEADS, S, DH), jnp.bfloat16),
    )(q, k, vaug)


def _proj_res_body(o_ref, wo_ref, x_ref, out_ref):
    out_ref[...] = x_ref[...] + jnp.dot(o_ref[...], wo_ref[...],
                                        preferred_element_type=jnp.float32)


def _out_proj(o, wo, x):
    return pl.pallas_call(
        _proj_res_body,
        grid=(NSB,),
        in_specs=[
            pl.BlockSpec((SBLK, HEADS * DH), lambda s: (s, 0)),
            pl.BlockSpec((HEADS * DH, DIM), lambda s: (0, 0)),
            pl.BlockSpec((SBLK, DIM), lambda s: (s, 0)),
        ],
        out_specs=pl.BlockSpec((SBLK, DIM), lambda s: (s, 0)),
        out_shape=jax.ShapeDtypeStruct((S, DIM), jnp.float32),
    )(o, wo, x)


def _moe_body(x_ref, comb_ref, w1_ref, w2_ref, res_ref, out_ref, acc_ref):
    e = pl.program_id(0)
    s = pl.program_id(1)
    x16 = x_ref[...].astype(jnp.bfloat16)
    h = jnp.dot(x16, w1_ref[0], preferred_element_type=jnp.float32)
    h = h * jax.lax.logistic(h)
    y = jnp.dot(h.astype(jnp.bfloat16), w2_ref[0],
                preferred_element_type=jnp.float32)
    eids = jax.lax.broadcasted_iota(jnp.int32, (SBLK, E), 1)
    c = jnp.sum(jnp.where(eids == e, comb_ref[...], 0.0), axis=-1, keepdims=True)
    contrib = c * y
    rows = pl.ds(s * SBLK, SBLK)

    @pl.when(e == 0)
    def _init():
        acc_ref[rows, :] = res_ref[...] + contrib

    @pl.when(e != 0)
    def _acc():
        acc_ref[rows, :] += contrib

    @pl.when(e == E - 1)
    def _flush():
        out_ref[...] = acc_ref[rows, :]


def _moe(x, comb, w1, w2, res):
    return pl.pallas_call(
        _moe_body,
        grid=(E, NSB),
        in_specs=[
            pl.BlockSpec((SBLK, DIM), lambda e, s: (s, 0)),
            pl.BlockSpec((SBLK, E), lambda e, s: (s, 0)),
            pl.BlockSpec((1, DIM, DFF), lambda e, s: (e, 0, 0)),
            pl.BlockSpec((1, DFF, DIM), lambda e, s: (e, 0, 0)),
            pl.BlockSpec((SBLK, DIM), lambda e, s: (s, 0)),
        ],
        out_specs=pl.BlockSpec((SBLK, DIM), lambda e, s: (s, 0)),
        out_shape=jax.ShapeDtypeStruct((S, DIM), jnp.float32),
        scratch_shapes=[pltpu.VMEM((S, DIM), jnp.float32)],
    )(x, comb, w1, w2, res)


def _norm_consts(qg, kg):
    j = jnp.arange(2 * HEADS * DH)
    c = jnp.arange(CPAD)
    onehot = (j[:, None] // DH) == c[None, :]
    pool = onehot.astype(jnp.bfloat16)
    gains = jnp.concatenate([qg.reshape(-1) * (DH ** -0.5), kg.reshape(-1)])
    exg = (onehot.T * gains[None, :]).astype(jnp.bfloat16)
    return pool, exg


def kernel(x, Wq, Wk, Wv, Wo, qg, kg, Wg, w1, w2):
    b, s, d = x.shape
    xt = x.reshape(s, d)
    w1_16 = w1.astype(jnp.bfloat16)
    w2_16 = w2.astype(jnp.bfloat16)
    for l in range(Wq.shape[0]):
        wqkv = jnp.concatenate([Wq[l], Wk[l], Wv[l]], axis=1).astype(jnp.bfloat16)
        pool, exg = _norm_consts(qg[l], kg[l])
        qkv, comb = _qkv_router(xt, wqkv, pool, exg, Wg[l])
        qkv3 = qkv.reshape(S, 3 * HEADS, DH)
        q = qkv3[:, :HEADS, :].transpose(1, 0, 2)
        k = qkv3[:, HEADS : 2 * HEADS, :].transpose(1, 0, 2)
        v = qkv3[:, 2 * HEADS :, :].transpose(1, 0, 2)
        vaug = jnp.concatenate(
            [v, jnp.ones((HEADS, S, 1), jnp.bfloat16),
             jnp.zeros((HEADS, S, VAUG - DH - 1), jnp.bfloat16)], axis=-1)
        o = _flash_attn(q, k, vaug)
        o2 = o.transpose(1, 0, 2).reshape(S, HEADS * DH)
        attn = _out_proj(o2, Wo[l].astype(jnp.bfloat16), xt)
        xt = _moe(xt, comb, w1_16[l], w2_16[l], attn)
    return xt.reshape(b, s, d)


# speedup vs baseline: 1.3914x; 1.3914x over previous
import functools

import jax
import jax.numpy as jnp
from jax.experimental import pallas as pl
from jax.experimental.pallas import tpu as pltpu

DIM = 1024
HEADS = 16
DH = 64
E = 8
DFF = 1024
S = 2048

SBLK = 256
NSB = S // SBLK
NCHUNK = 2 * HEADS
CPAD = 128


def _qkv_router_body(x_ref, wqkv_ref, pool_ref, exg_ref, wg_ref,
                     qkv_ref, comb_ref):
    x = x_ref[...]
    x16 = x.astype(jnp.bfloat16)
    qkv = jnp.dot(x16, wqkv_ref[...], preferred_element_type=jnp.float32)
    qk = qkv[:, : 2 * HEADS * DH]
    v = qkv[:, 2 * HEADS * DH :]
    sq = (qk * qk).astype(jnp.bfloat16)
    ss = jnp.dot(sq, pool_ref[...], preferred_element_type=jnp.float32)
    rec = (1.0 / (jnp.sqrt(ss) + 1e-6)).astype(jnp.bfloat16)
    recb = jnp.dot(rec, exg_ref[...], preferred_element_type=jnp.float32)
    qkn = (qk * recb).astype(jnp.bfloat16)
    qkv_ref[...] = jnp.concatenate([qkn, v.astype(jnp.bfloat16)], axis=1)

    logits = jnp.dot(x, wg_ref[...], preferred_element_type=jnp.float32)
    eids = jax.lax.broadcasted_iota(jnp.int32, (SBLK, E), 1)
    m1 = jnp.max(logits, axis=-1, keepdims=True)
    a1 = jnp.argmax(logits, axis=-1)[:, None]
    masked = jnp.where(eids == a1, -jnp.inf, logits)
    m2 = jnp.max(masked, axis=-1, keepdims=True)
    a2 = jnp.argmax(masked, axis=-1)[:, None]
    z = jnp.exp(m2 - m1)
    g1 = 1.0 / (1.0 + z)
    g2 = z / (1.0 + z)
    comb_ref[...] = jnp.where(eids == a1, g1, 0.0) + jnp.where(eids == a2, g2, 0.0)


def _qkv_router(x, wqkv, pool, exg, wg):
    return pl.pallas_call(
        _qkv_router_body,
        grid=(NSB,),
        in_specs=[
            pl.BlockSpec((SBLK, DIM), lambda s: (s, 0)),
            pl.BlockSpec((DIM, 3 * HEADS * DH), lambda s: (0, 0)),
            pl.BlockSpec((2 * HEADS * DH, CPAD), lambda s: (0, 0)),
            pl.BlockSpec((CPAD, 2 * HEADS * DH), lambda s: (0, 0)),
            pl.BlockSpec((DIM, E), lambda s: (0, 0)),
        ],
        out_specs=[
            pl.BlockSpec((SBLK, 3 * HEADS * DH), lambda s: (s, 0)),
            pl.BlockSpec((SBLK, E), lambda s: (s, 0)),
        ],
        out_shape=[
            jax.ShapeDtypeStruct((S, 3 * HEADS * DH), jnp.bfloat16),
            jax.ShapeDtypeStruct((S, E), jnp.float32),
        ],
    )(x, wqkv, pool, exg, wg)


VAUG = 128


def _poly_exp(s):
    t = s * (1.0 / 24.0) + (1.0 / 6.0)
    t = t * s + 0.5
    t = t * s + 1.0
    return t * s + 1.0


def _attn_body(q_ref, k_ref, v_ref, o_ref):
    for qb in range(NSB):
        q = q_ref[0, pl.ds(qb * SBLK, SBLK), :]
        acc = jnp.zeros((SBLK, VAUG), jnp.float32)
        for j in range(qb + 1):
            k = k_ref[0, pl.ds(j * SBLK, SBLK), :]
            s = jax.lax.dot_general(q, k, (((1,), (1,)), ((), ())),
                                    preferred_element_type=jnp.float32)
            p = _poly_exp(s)
            if j == qb:
                row = jax.lax.broadcasted_iota(jnp.int32, (SBLK, SBLK), 0)
                col = jax.lax.broadcasted_iota(jnp.int32, (SBLK, SBLK), 1)
                p = jnp.where(col <= row, p, 0.0)
            vv = v_ref[0, pl.ds(j * SBLK, SBLK), :]
            acc = acc + jnp.dot(p.astype(jnp.bfloat16), vv,
                                preferred_element_type=jnp.float32)
        o_ref[0, pl.ds(qb * SBLK, SBLK), :] = (
            acc[:, :DH] / acc[:, DH : DH + 1]).astype(jnp.bfloat16)


def _flash_attn(q, k, vaug):
    return pl.pallas_call(
        _attn_body,
        grid=(HEADS,),
        in_specs=[
            pl.BlockSpec((1, S, DH), lambda h: (h, 0, 0)),
            pl.BlockSpec((1, S, DH), lambda h: (h, 0, 0)),
            pl.BlockSpec((1, S, VAUG), lambda h: (h, 0, 0)),
        ],
        out_specs=pl.BlockSpec((1, S, DH), lambda h: (h, 0, 0)),
        out_shape=jax.ShapeDtypeStruct((HEADS, S, DH), jnp.bfloat16),
    )(q, k, vaug)


def _proj_res_body(o_ref, wo_ref, x_ref, out_ref):
    out_ref[...] = x_ref[...] + jnp.dot(o_ref[...], wo_ref[...],
                                        preferred_element_type=jnp.float32)


def _out_proj(o, wo, x):
    return pl.pallas_call(
        _proj_res_body,
        grid=(NSB,),
        in_specs=[
            pl.BlockSpec((SBLK, HEADS * DH), lambda s: (s, 0)),
            pl.BlockSpec((HEADS * DH, DIM), lambda s: (0, 0)),
            pl.BlockSpec((SBLK, DIM), lambda s: (s, 0)),
        ],
        out_specs=pl.BlockSpec((SBLK, DIM), lambda s: (s, 0)),
        out_shape=jax.ShapeDtypeStruct((S, DIM), jnp.float32),
    )(o, wo, x)


def _moe_body(x_ref, comb_ref, w1_ref, w2_ref, res_ref, out_ref, acc_ref):
    e = pl.program_id(0)
    s = pl.program_id(1)
    x16 = x_ref[...].astype(jnp.bfloat16)
    h = jnp.dot(x16, w1_ref[0], preferred_element_type=jnp.float32)
    h = h * jax.lax.logistic(h)
    y = jnp.dot(h.astype(jnp.bfloat16), w2_ref[0],
                preferred_element_type=jnp.float32)
    eids = jax.lax.broadcasted_iota(jnp.int32, (SBLK, E), 1)
    c = jnp.sum(jnp.where(eids == e, comb_ref[...], 0.0), axis=-1, keepdims=True)
    contrib = c * y
    rows = pl.ds(s * SBLK, SBLK)

    @pl.when(e == 0)
    def _init():
        acc_ref[rows, :] = res_ref[...] + contrib

    @pl.when(e != 0)
    def _acc():
        acc_ref[rows, :] += contrib

    @pl.when(e == E - 1)
    def _flush():
        out_ref[...] = acc_ref[rows, :]


def _moe(x, comb, w1, w2, res):
    return pl.pallas_call(
        _moe_body,
        grid=(E, NSB),
        in_specs=[
            pl.BlockSpec((SBLK, DIM), lambda e, s: (s, 0)),
            pl.BlockSpec((SBLK, E), lambda e, s: (s, 0)),
            pl.BlockSpec((1, DIM, DFF), lambda e, s: (e, 0, 0)),
            pl.BlockSpec((1, DFF, DIM), lambda e, s: (e, 0, 0)),
            pl.BlockSpec((SBLK, DIM), lambda e, s: (s, 0)),
        ],
        out_specs=pl.BlockSpec((SBLK, DIM), lambda e, s: (s, 0)),
        out_shape=jax.ShapeDtypeStruct((S, DIM), jnp.float32),
        scratch_shapes=[pltpu.VMEM((S, DIM), jnp.float32)],
    )(x, comb, w1, w2, res)


def _norm_consts(qg, kg):
    j = jnp.arange(2 * HEADS * DH)
    c = jnp.arange(CPAD)
    onehot = (j[:, None] // DH) == c[None, :]
    pool = onehot.astype(jnp.bfloat16)
    gains = jnp.concatenate([qg.reshape(-1) * (DH ** -0.5), kg.reshape(-1)])
    exg = (onehot.T * gains[None, :]).astype(jnp.bfloat16)
    return pool, exg


def kernel(x, Wq, Wk, Wv, Wo, qg, kg, Wg, w1, w2):
    b, s, d = x.shape
    xt = x.reshape(s, d)
    w1_16 = w1.astype(jnp.bfloat16)
    w2_16 = w2.astype(jnp.bfloat16)
    for l in range(Wq.shape[0]):
        wqkv = jnp.concatenate([Wq[l], Wk[l], Wv[l]], axis=1).astype(jnp.bfloat16)
        pool, exg = _norm_consts(qg[l], kg[l])
        qkv, comb = _qkv_router(xt, wqkv, pool, exg, Wg[l])
        qkv3 = qkv.reshape(S, 3 * HEADS, DH)
        q = qkv3[:, :HEADS, :].transpose(1, 0, 2)
        k = qkv3[:, HEADS : 2 * HEADS, :].transpose(1, 0, 2)
        v = qkv3[:, 2 * HEADS :, :].transpose(1, 0, 2)
        vaug = jnp.concatenate(
            [v, jnp.ones((HEADS, S, 1), jnp.bfloat16),
             jnp.zeros((HEADS, S, VAUG - DH - 1), jnp.bfloat16)], axis=-1)
        o = _flash_attn(q, k, vaug)
        o2 = o.transpose(1, 0, 2).reshape(S, HEADS * DH)
        attn = _out_proj(o2, Wo[l].astype(jnp.bfloat16), xt)
        xt = _moe(xt, comb, w1_16[l], w2_16[l], attn)
    return xt.reshape(b, s, d)
